# baseline (speedup 1.0000x reference)
import jax
import jax.numpy as jnp
from jax.experimental import pallas as pl

NUM_LATENTS = 8192
LATENT_DIM = 32
TM = 256


def _norm_rows(v):
    return v / (jnp.sqrt(jnp.sum(v * v, axis=-1, keepdims=True)) + 1e-08)


def _vq_tile(x_ref, cbp_ref, zq_ref, z_ref, xn_ref, idx_ref):
    x = x_ref[...]
    cbp = cbp_ref[...]
    cb = _norm_rows(cbp)
    cbn = _norm_rows(cb)
    xn = _norm_rows(x)
    s = jax.lax.dot_general(
        xn, cbn, (((1,), (1,)), ((), ())),
        preferred_element_type=jnp.float32)
    idx = jnp.argmax(s, axis=1).astype(jnp.int32)
    oh = (jax.lax.broadcasted_iota(jnp.int32, (TM, NUM_LATENTS), 1)
          == idx[:, None]).astype(jnp.float32)
    z = jax.lax.dot_general(
        oh, cb, (((1,), (0,)), ((), ())),
        preferred_element_type=jnp.float32,
        precision=jax.lax.Precision.HIGHEST)
    zq_ref[...] = xn + (z - xn)
    z_ref[...] = z
    xn_ref[...] = xn
    idx_ref[0, 0, :] = idx


def kernel(x, codebook_param, training):
    B, N, D = x.shape
    T = B * N
    nt = T // TM
    xf = x.reshape(T, D)
    zq, z, xn, idx = pl.pallas_call(
        _vq_tile,
        grid=(nt,),
        in_specs=[
            pl.BlockSpec((TM, D), lambda i: (i, 0)),
            pl.BlockSpec((NUM_LATENTS, D), lambda i: (0, 0)),
        ],
        out_specs=[
            pl.BlockSpec((TM, D), lambda i: (i, 0)),
            pl.BlockSpec((TM, D), lambda i: (i, 0)),
            pl.BlockSpec((TM, D), lambda i: (i, 0)),
            pl.BlockSpec((1, 1, TM), lambda i: (i, 0, 0)),
        ],
        out_shape=[
            jax.ShapeDtypeStruct((T, D), jnp.float32),
            jax.ShapeDtypeStruct((T, D), jnp.float32),
            jax.ShapeDtypeStruct((T, D), jnp.float32),
            jax.ShapeDtypeStruct((nt, 1, TM), jnp.int32),
        ],
    )(xf, codebook_param)
    return (zq.reshape(B, N, D), z.reshape(B, N, D),
            xn.reshape(B, N, D), idx.reshape(B, N))


# speedup vs baseline: 5.3734x; 5.3734x over previous
import functools

import jax
import jax.numpy as jnp
from jax import lax
from jax.experimental import pallas as pl
from jax.experimental.pallas import tpu as pltpu
from jax.experimental.pallas import tpu_sc as plsc

NUM_LATENTS = 8192
LATENT_DIM = 32
TOKENS = 16 * 1024
TM = 256

_NC, _NS = 2, 16
_NW = _NC * _NS
_B_PER_W = TOKENS // _NW


def _norm_rows(v):
    return v / (jnp.sqrt(jnp.sum(v * v, axis=-1, keepdims=True)) + 1e-08)


def _vq_main(x_ref, cbp_ref, xn_ref, idx_ref, cb_ref, cbn_scr):
    @pl.when(pl.program_id(0) == 0)
    def _():
        cb = _norm_rows(cbp_ref[...])
        cb_ref[...] = cb
        cbn_scr[...] = _norm_rows(cb)

    xn = _norm_rows(x_ref[...])
    s = jax.lax.dot_general(
        xn, cbn_scr[...], (((1,), (1,)), ((), ())),
        preferred_element_type=jnp.float32)
    idx_ref[0, 0, :] = jnp.argmax(s, axis=1).astype(jnp.int32)
    xn_ref[...] = xn


_sc_mesh = plsc.VectorSubcoreMesh(core_axis_name="c", subcore_axis_name="s")


_GW = 128


@functools.partial(
    pl.kernel, mesh=_sc_mesh,
    out_type=jax.ShapeDtypeStruct((TOKENS, _GW), jnp.float32),
    scratch_types=[
        pltpu.VMEM((_B_PER_W,), jnp.int32),
        pltpu.VMEM((_B_PER_W, _GW), jnp.float32),
        pltpu.SemaphoreType.DMA,
    ],
)
def _sc_gather(table_hbm, idx_hbm, out_hbm, idx_v, rows_v, sem):
    wid = lax.axis_index("s") * _NC + lax.axis_index("c")
    base = wid * _B_PER_W
    pltpu.sync_copy(idx_hbm.at[pl.ds(base, _B_PER_W)], idx_v)
    pltpu.async_copy(table_hbm.at[idx_v], rows_v, sem).wait()
    pltpu.sync_copy(rows_v, out_hbm.at[pl.ds(base, _B_PER_W)])


def kernel(x, codebook_param, training):
    B, N, D = x.shape
    T = B * N
    nt = T // TM
    xf = x.reshape(T, D)
    xn, idx, cb = pl.pallas_call(
        _vq_main,
        grid=(nt,),
        in_specs=[
            pl.BlockSpec((TM, D), lambda i: (i, 0)),
            pl.BlockSpec((NUM_LATENTS, D), lambda i: (0, 0)),
        ],
        out_specs=[
            pl.BlockSpec((TM, D), lambda i: (i, 0)),
            pl.BlockSpec((1, 1, TM), lambda i: (i, 0, 0)),
            pl.BlockSpec((NUM_LATENTS, D), lambda i: (0, 0)),
        ],
        out_shape=[
            jax.ShapeDtypeStruct((T, D), jnp.float32),
            jax.ShapeDtypeStruct((nt, 1, TM), jnp.int32),
            jax.ShapeDtypeStruct((NUM_LATENTS, D), jnp.float32),
        ],
        scratch_shapes=[pltpu.VMEM((NUM_LATENTS, D), jnp.float32)],
    )(xf, codebook_param)
    idx_flat = idx.reshape(T)
    cb_pad = jnp.pad(cb, ((0, 0), (0, _GW - D)))
    z = _sc_gather(cb_pad, idx_flat)[:, :D]
    return (z.reshape(B, N, D), z.reshape(B, N, D),
            xn.reshape(B, N, D), idx_flat.reshape(B, N))


# speedup vs baseline: 5.3983x; 1.0046x over previous
import functools

import jax
import jax.numpy as jnp
from jax import lax
from jax.experimental import pallas as pl
from jax.experimental.pallas import tpu as pltpu
from jax.experimental.pallas import tpu_sc as plsc

NUM_LATENTS = 8192
LATENT_DIM = 32
TOKENS = 16 * 1024
TM = 256

_NC, _NS = 2, 16
_NW = _NC * _NS
_B_PER_W = TOKENS // _NW

_GW = 128


def _norm_rows(v):
    return v / (jnp.sqrt(jnp.sum(v * v, axis=-1, keepdims=True)) + 1e-08)


def _cb_prep(cbp_ref, cbn_ref, cbpad_ref):
    cb = _norm_rows(cbp_ref[...])
    cbn_ref[...] = _norm_rows(cb)
    cbpad_ref[...] = jnp.concatenate(
        [cb, jnp.zeros((NUM_LATENTS, _GW - LATENT_DIM), jnp.float32)], axis=1)


def _vq_main(x_ref, cbn_ref, xn_ref, idx_ref):
    xn = _norm_rows(x_ref[...])
    s = jax.lax.dot_general(
        xn, cbn_ref[...], (((1,), (1,)), ((), ())),
        preferred_element_type=jnp.float32)
    idx_ref[0, 0, :] = jnp.argmax(s, axis=1).astype(jnp.int32)
    xn_ref[...] = xn


_sc_mesh = plsc.VectorSubcoreMesh(core_axis_name="c", subcore_axis_name="s")


@functools.partial(
    pl.kernel, mesh=_sc_mesh,
    out_type=jax.ShapeDtypeStruct((TOKENS, _GW), jnp.float32),
    scratch_types=[
        pltpu.VMEM((_B_PER_W,), jnp.int32),
        pltpu.VMEM((_B_PER_W, _GW), jnp.float32),
        pltpu.SemaphoreType.DMA,
    ],
)
def _sc_gather(table_hbm, idx_hbm, out_hbm, idx_v, rows_v, sem):
    wid = lax.axis_index("s") * _NC + lax.axis_index("c")
    base = wid * _B_PER_W
    pltpu.sync_copy(idx_hbm.at[pl.ds(base, _B_PER_W)], idx_v)
    pltpu.async_copy(table_hbm.at[idx_v], rows_v, sem).wait()
    pltpu.sync_copy(rows_v, out_hbm.at[pl.ds(base, _B_PER_W)])


def kernel(x, codebook_param, training):
    B, N, D = x.shape
    T = B * N
    nt = T // TM
    xf = x.reshape(T, D)
    cbn, cb_pad = pl.pallas_call(
        _cb_prep,
        in_specs=[pl.BlockSpec((NUM_LATENTS, D), lambda: (0, 0))],
        out_specs=[
            pl.BlockSpec((NUM_LATENTS, D), lambda: (0, 0)),
            pl.BlockSpec((NUM_LATENTS, _GW), lambda: (0, 0)),
        ],
        out_shape=[
            jax.ShapeDtypeStruct((NUM_LATENTS, D), jnp.float32),
            jax.ShapeDtypeStruct((NUM_LATENTS, _GW), jnp.float32),
        ],
    )(codebook_param)
    xn, idx = pl.pallas_call(
        _vq_main,
        grid=(nt,),
        in_specs=[
            pl.BlockSpec((TM, D), lambda i: (i, 0)),
            pl.BlockSpec((NUM_LATENTS, D), lambda i: (0, 0)),
        ],
        out_specs=[
            pl.BlockSpec((TM, D), lambda i: (i, 0)),
            pl.BlockSpec((1, 1, TM), lambda i: (i, 0, 0)),
        ],
        out_shape=[
            jax.ShapeDtypeStruct((T, D), jnp.float32),
            jax.ShapeDtypeStruct((nt, 1, TM), jnp.int32),
        ],
        compiler_params=pltpu.CompilerParams(
            dimension_semantics=("parallel",)),
    )(xf, cbn)
    idx_flat = idx.reshape(T)
    z = _sc_gather(cb_pad, idx_flat)[:, :D]
    return (z.reshape(B, N, D), z.reshape(B, N, D),
            xn.reshape(B, N, D), idx_flat.reshape(B, N))


# speedup vs baseline: 5.7496x; 1.0651x over previous
import functools

import jax
import jax.numpy as jnp
from jax import lax
from jax.experimental import pallas as pl
from jax.experimental.pallas import tpu as pltpu
from jax.experimental.pallas import tpu_sc as plsc

NUM_LATENTS = 8192
LATENT_DIM = 32
TOKENS = 16 * 1024
TM = 512

_NC, _NS = 2, 16
_NW = _NC * _NS
_B_PER_W = TOKENS // _NW

_GW = 128


def _norm_rows(v):
    return v / (jnp.sqrt(jnp.sum(v * v, axis=-1, keepdims=True)) + 1e-08)


def _cb_prep(cbp_ref, cbn_ref, cbpad_ref):
    cb = _norm_rows(cbp_ref[...])
    cbn_ref[...] = _norm_rows(cb)
    cbpad_ref[...] = jnp.concatenate(
        [cb, jnp.zeros((NUM_LATENTS, _GW - LATENT_DIM), jnp.float32)], axis=1)


def _vq_main(x_ref, cbn_ref, xn_ref, idx_ref):
    xn = _norm_rows(x_ref[...])
    s = jax.lax.dot_general(
        xn, cbn_ref[...], (((1,), (1,)), ((), ())),
        preferred_element_type=jnp.float32)
    idx_ref[0, 0, :] = jnp.argmax(s, axis=1).astype(jnp.int32)
    xn_ref[...] = xn


_sc_mesh = plsc.VectorSubcoreMesh(core_axis_name="c", subcore_axis_name="s")


@functools.partial(
    pl.kernel, mesh=_sc_mesh,
    out_type=jax.ShapeDtypeStruct((TOKENS, _GW), jnp.float32),
    scratch_types=[
        pltpu.VMEM((_B_PER_W,), jnp.int32),
        pltpu.VMEM((_B_PER_W, _GW), jnp.float32),
        pltpu.SemaphoreType.DMA,
    ],
)
def _sc_gather(table_hbm, idx_hbm, out_hbm, idx_v, rows_v, sem):
    wid = lax.axis_index("s") * _NC + lax.axis_index("c")
    base = wid * _B_PER_W
    pltpu.sync_copy(idx_hbm.at[pl.ds(base, _B_PER_W)], idx_v)
    pltpu.async_copy(table_hbm.at[idx_v], rows_v, sem).wait()
    pltpu.sync_copy(rows_v, out_hbm.at[pl.ds(base, _B_PER_W)])


def kernel(x, codebook_param, training):
    B, N, D = x.shape
    T = B * N
    nt = T // TM
    xf = x.reshape(T, D)
    cbn, cb_pad = pl.pallas_call(
        _cb_prep,
        in_specs=[pl.BlockSpec((NUM_LATENTS, D), lambda: (0, 0))],
        out_specs=[
            pl.BlockSpec((NUM_LATENTS, D), lambda: (0, 0)),
            pl.BlockSpec((NUM_LATENTS, _GW), lambda: (0, 0)),
        ],
        out_shape=[
            jax.ShapeDtypeStruct((NUM_LATENTS, D), jnp.float32),
            jax.ShapeDtypeStruct((NUM_LATENTS, _GW), jnp.float32),
        ],
    )(codebook_param)
    xn, idx = pl.pallas_call(
        _vq_main,
        grid=(nt,),
        in_specs=[
            pl.BlockSpec((TM, D), lambda i: (i, 0)),
            pl.BlockSpec((NUM_LATENTS, D), lambda i: (0, 0)),
        ],
        out_specs=[
            pl.BlockSpec((TM, D), lambda i: (i, 0)),
            pl.BlockSpec((1, 1, TM), lambda i: (i, 0, 0)),
        ],
        out_shape=[
            jax.ShapeDtypeStruct((T, D), jnp.float32),
            jax.ShapeDtypeStruct((nt, 1, TM), jnp.int32),
        ],
        compiler_params=pltpu.CompilerParams(
            dimension_semantics=("parallel",)),
    )(xf, cbn)
    idx_flat = idx.reshape(T)
    z = _sc_gather(cb_pad, idx_flat)[:, :D]
    return (z.reshape(B, N, D), z.reshape(B, N, D),
            xn.reshape(B, N, D), idx_flat.reshape(B, N))


# speedup vs baseline: 6.2870x; 1.0935x over previous
import functools

import jax
import jax.numpy as jnp
from jax import lax
from jax.experimental import pallas as pl
from jax.experimental.pallas import tpu as pltpu
from jax.experimental.pallas import tpu_sc as plsc

NUM_LATENTS = 8192
LATENT_DIM = 32
TOKENS = 16 * 1024
TM = 4096

_NC, _NS = 2, 16
_NW = _NC * _NS
_B_PER_W = TOKENS // _NW

_GW = 128


def _norm_rows(v):
    return v / (jnp.sqrt(jnp.sum(v * v, axis=-1, keepdims=True)) + 1e-08)


def _cb_prep(cbp_ref, cbn_ref, cbpad_ref):
    cb = _norm_rows(cbp_ref[...])
    cbn_ref[...] = _norm_rows(cb)
    cbpad_ref[...] = jnp.concatenate(
        [cb, jnp.zeros((NUM_LATENTS, _GW - LATENT_DIM), jnp.float32)], axis=1)


SUB = 256


def _vq_main(x_ref, cbn_ref, xn_ref, idx_ref):
    xn = _norm_rows(x_ref[...])
    cbn = cbn_ref[...]
    for j in range(TM // SUB):
        s = jax.lax.dot_general(
            xn[j * SUB:(j + 1) * SUB, :], cbn, (((1,), (1,)), ((), ())),
            preferred_element_type=jnp.float32)
        idx_ref[0, 0, j * SUB:(j + 1) * SUB] = (
            jnp.argmax(s, axis=1).astype(jnp.int32))
    xn_ref[...] = xn


_sc_mesh = plsc.VectorSubcoreMesh(core_axis_name="c", subcore_axis_name="s")


_HW = _B_PER_W // 2


@functools.partial(
    pl.kernel, mesh=_sc_mesh,
    out_type=jax.ShapeDtypeStruct((TOKENS, _GW), jnp.float32),
    scratch_types=[
        pltpu.VMEM((_B_PER_W,), jnp.int32),
        pltpu.VMEM((_HW, _GW), jnp.float32),
        pltpu.VMEM((_HW, _GW), jnp.float32),
        pltpu.SemaphoreType.DMA,
        pltpu.SemaphoreType.DMA,
    ],
)
def _sc_gather(table_hbm, idx_hbm, out_hbm, idx_v, rows_a, rows_b, sem_a,
               sem_b):
    wid = lax.axis_index("s") * _NC + lax.axis_index("c")
    base = wid * _B_PER_W
    pltpu.sync_copy(idx_hbm.at[pl.ds(base, _B_PER_W)], idx_v)
    ca = pltpu.async_copy(table_hbm.at[idx_v.at[pl.ds(0, _HW)]], rows_a, sem_a)
    cb = pltpu.async_copy(table_hbm.at[idx_v.at[pl.ds(_HW, _HW)]], rows_b,
                          sem_b)
    ca.wait()
    pltpu.sync_copy(rows_a, out_hbm.at[pl.ds(base, _HW)])
    cb.wait()
    pltpu.sync_copy(rows_b, out_hbm.at[pl.ds(base + _HW, _HW)])


def kernel(x, codebook_param, training):
    B, N, D = x.shape
    T = B * N
    nt = T // TM
    xf = x.reshape(T, D)
    cbn, cb_pad = pl.pallas_call(
        _cb_prep,
        in_specs=[pl.BlockSpec((NUM_LATENTS, D), lambda: (0, 0))],
        out_specs=[
            pl.BlockSpec((NUM_LATENTS, D), lambda: (0, 0)),
            pl.BlockSpec((NUM_LATENTS, _GW), lambda: (0, 0)),
        ],
        out_shape=[
            jax.ShapeDtypeStruct((NUM_LATENTS, D), jnp.float32),
            jax.ShapeDtypeStruct((NUM_LATENTS, _GW), jnp.float32),
        ],
    )(codebook_param)
    xn, idx = pl.pallas_call(
        _vq_main,
        grid=(nt,),
        in_specs=[
            pl.BlockSpec((TM, D), lambda i: (i, 0)),
            pl.BlockSpec((NUM_LATENTS, D), lambda i: (0, 0)),
        ],
        out_specs=[
            pl.BlockSpec((TM, D), lambda i: (i, 0)),
            pl.BlockSpec((1, 1, TM), lambda i: (i, 0, 0)),
        ],
        out_shape=[
            jax.ShapeDtypeStruct((T, D), jnp.float32),
            jax.ShapeDtypeStruct((nt, 1, TM), jnp.int32),
        ],
        compiler_params=pltpu.CompilerParams(
            dimension_semantics=("parallel",)),
    )(xf, cbn)
    idx_flat = idx.reshape(T)
    z = _sc_gather(cb_pad, idx_flat)[:, :D]
    return (z.reshape(B, N, D), z.reshape(B, N, D),
            xn.reshape(B, N, D), idx_flat.reshape(B, N))


# speedup vs baseline: 6.3283x; 1.0066x over previous
import functools

import jax
import jax.numpy as jnp
from jax import lax
from jax.experimental import pallas as pl
from jax.experimental.pallas import tpu as pltpu
from jax.experimental.pallas import tpu_sc as plsc

NUM_LATENTS = 8192
LATENT_DIM = 32
TOKENS = 16 * 1024
TM = 4096

_NC, _NS = 2, 16
_NW = _NC * _NS
_B_PER_W = TOKENS // _NW

_GW = 128


def _norm_rows(v):
    return v / (jnp.sqrt(jnp.sum(v * v, axis=-1, keepdims=True)) + 1e-08)


def _cb_prep(cbp_ref, cbn_ref, cbpad_ref):
    cb = _norm_rows(cbp_ref[...])
    cbn_ref[...] = _norm_rows(cb)
    cbpad_ref[...] = jnp.concatenate(
        [cb, jnp.zeros((NUM_LATENTS, _GW - LATENT_DIM), jnp.float32)], axis=1)


SUB = 256


def _vq_main(x_ref, cbn_ref, xn_ref, idx_ref):
    xn = _norm_rows(x_ref[...])
    cbn = cbn_ref[...]
    for j in range(TM // SUB):
        s = jax.lax.dot_general(
            xn[j * SUB:(j + 1) * SUB, :], cbn, (((1,), (1,)), ((), ())),
            preferred_element_type=jnp.float32)
        idx_ref[0, 0, j * SUB:(j + 1) * SUB] = (
            jnp.argmax(s, axis=1).astype(jnp.int32))
    xn_ref[...] = xn


_sc_mesh = plsc.VectorSubcoreMesh(core_axis_name="c", subcore_axis_name="s")


@functools.partial(
    pl.kernel, mesh=_sc_mesh,
    out_type=jax.ShapeDtypeStruct((TOKENS, _GW), jnp.float32),
    scratch_types=[
        pltpu.VMEM((_B_PER_W,), jnp.int32),
        pltpu.VMEM((_B_PER_W, _GW), jnp.float32),
        pltpu.SemaphoreType.DMA,
    ],
)
def _sc_gather(table_hbm, idx_hbm, out_hbm, idx_v, rows_v, sem):
    wid = lax.axis_index("s") * _NC + lax.axis_index("c")
    base = wid * _B_PER_W
    pltpu.sync_copy(idx_hbm.at[pl.ds(base, _B_PER_W)], idx_v)
    pltpu.async_copy(table_hbm.at[idx_v], rows_v, sem).wait()
    pltpu.sync_copy(rows_v, out_hbm.at[pl.ds(base, _B_PER_W)])


def kernel(x, codebook_param, training):
    B, N, D = x.shape
    T = B * N
    nt = T // TM
    xf = x.reshape(T, D)
    cbn, cb_pad = pl.pallas_call(
        _cb_prep,
        in_specs=[pl.BlockSpec((NUM_LATENTS, D), lambda: (0, 0))],
        out_specs=[
            pl.BlockSpec((NUM_LATENTS, D), lambda: (0, 0)),
            pl.BlockSpec((NUM_LATENTS, _GW), lambda: (0, 0)),
        ],
        out_shape=[
            jax.ShapeDtypeStruct((NUM_LATENTS, D), jnp.float32),
            jax.ShapeDtypeStruct((NUM_LATENTS, _GW), jnp.float32),
        ],
    )(codebook_param)
    xn, idx = pl.pallas_call(
        _vq_main,
        grid=(nt,),
        in_specs=[
            pl.BlockSpec((TM, D), lambda i: (i, 0)),
            pl.BlockSpec((NUM_LATENTS, D), lambda i: (0, 0)),
        ],
        out_specs=[
            pl.BlockSpec((TM, D), lambda i: (i, 0)),
            pl.BlockSpec((1, 1, TM), lambda i: (i, 0, 0)),
        ],
        out_shape=[
            jax.ShapeDtypeStruct((T, D), jnp.float32),
            jax.ShapeDtypeStruct((nt, 1, TM), jnp.int32),
        ],
        compiler_params=pltpu.CompilerParams(
            dimension_semantics=("parallel",)),
    )(xf, cbn)
    idx_flat = idx.reshape(T)
    z = _sc_gather(cb_pad, idx_flat)[:, :D]
    return (z.reshape(B, N, D), z.reshape(B, N, D),
            xn.reshape(B, N, D), idx_flat.reshape(B, N))
